# baseline (speedup 1.0000x reference)
import jax
import jax.numpy as jnp
from jax import lax
from jax.experimental import pallas as pl
from jax.experimental.pallas import tpu as pltpu
from jax.experimental.pallas import tpu_sc as plsc

LANES = 16


def _make_kernel(B, H, V, D):
    info = plsc.get_sparse_core_info()
    NC, NS = info.num_cores, info.num_subcores
    NW = NC * NS
    EPW = B // NW
    C = 4
    NCH = EPW // C
    FL = C * H
    G0 = 128
    G1 = H - G0
    NLG = H // LANES
    TAIL = H - NLG * LANES
    TOFF = H - LANES
    ND = D // LANES

    mesh = plsc.VectorSubcoreMesh(core_axis_name="c", subcore_axis_name="s")

    def body(feat_hbm, wts_hbm, table_hbm, out_hbm,
             idx0, idx1, w0, w1, rows0, rows1, out_v,
             sem_f0, sem_f1, sem_w0, sem_w1, sem_g0, sem_g1):
        cid = lax.axis_index("c")
        sid = lax.axis_index("s")
        wid = sid * NC + cid
        row0 = wid * EPW

        slots = (
            (idx0, w0, rows0, sem_f0, sem_w0, sem_g0),
            (idx1, w1, rows1, sem_f1, sem_w1, sem_g1),
        )

        def issue_in(k, s):
            idx_v, w_v, _, sem_f, sem_w, _ = slots[s]
            r = row0 + k * C
            pltpu.async_copy(feat_hbm.at[pl.ds(r, C)], idx_v, sem_f)
            pltpu.async_copy(wts_hbm.at[pl.ds(r, C)], w_v, sem_w)

        def wait_in(s):
            idx_v, w_v, _, sem_f, sem_w, _ = slots[s]
            pltpu.make_async_copy(feat_hbm.at[pl.ds(0, C)], idx_v, sem_f).wait()
            pltpu.make_async_copy(wts_hbm.at[pl.ds(0, C)], w_v, sem_w).wait()

        def issue_gathers(s):
            idx_v, _, rows_v, _, _, sem_g = slots[s]
            for i in range(C):
                pltpu.async_copy(
                    table_hbm.at[idx_v.at[i].at[pl.ds(0, G0)]],
                    rows_v.at[pl.ds(i * H, G0)],
                    sem_g,
                )
                pltpu.async_copy(
                    table_hbm.at[idx_v.at[i].at[pl.ds(G0, G1)]],
                    rows_v.at[pl.ds(i * H + G0, G1)],
                    sem_g,
                )

        def double_indices(s):
            lane = lax.broadcasted_iota(jnp.int32, (LANES,), 0)
            tail_mult = jnp.where(lane >= LANES - TAIL, 2, 1)
            idx_v = slots[s][0]
            for i in range(C):
                row = idx_v.at[i]
                for g in range(NLG):
                    row[pl.ds(g * LANES, LANES)] = row[pl.ds(g * LANES, LANES)] * 2
                row[pl.ds(TOFF, LANES)] = row[pl.ds(TOFF, LANES)] * tail_mult

        def wait_gathers(s):
            _, _, rows_v, _, _, sem_g = slots[s]
            pltpu.make_async_copy(table_hbm.at[pl.ds(0, FL)], rows_v, sem_g).wait()

        def compute(k, s):
            _, w_v, rows_v, _, _, _ = slots[s]
            ev = lax.broadcasted_iota(jnp.int32, (LANES,), 0) * 2

            def axpy(accs, w_s, r):
                accs = list(accs)
                for p in range(ND // 2):
                    x = rows_v[r, pl.ds(p * 2 * LANES, 2 * LANES)]
                    a, b = plsc.unpack(x, format=plsc.PackFormat.INTERLEAVED)
                    accs[2 * p] = accs[2 * p] + w_s * a
                    accs[2 * p + 1] = accs[2 * p + 1] + w_s * b
                return tuple(accs)

            def elem_body(i, carry):
                base = i * H

                def lg_body(lg, accs):
                    off = lg * LANES
                    wv = w_v[i, pl.ds(off, LANES)]
                    for j in range(LANES):
                        accs = axpy(accs, wv[j], base + off + j)
                    return accs

                zero = jnp.zeros((LANES,), jnp.float32)
                accs = lax.fori_loop(0, NLG, lg_body, (zero,) * ND)
                wv = w_v[i, pl.ds(TOFF, LANES)]
                for j in range(TAIL):
                    jj = LANES - TAIL + j
                    accs = axpy(accs, wv[jj], base + TOFF + jj)
                row_i = jnp.full((LANES,), i, jnp.int32)
                for p in range(ND // 2):
                    plsc.store_scatter(out_v, [row_i, p * 2 * LANES + ev], accs[2 * p])
                    plsc.store_scatter(out_v, [row_i, p * 2 * LANES + ev + 1], accs[2 * p + 1])
                return carry

            lax.fori_loop(0, C, elem_body, 0)
            pltpu.sync_copy(out_v, out_hbm.at[pl.ds(row0 + k * C, C)])

        issue_in(0, 0)
        issue_in(1, 1)
        wait_in(0)
        double_indices(0)
        issue_gathers(0)

        def outer(c2, carry):
            for par in range(2):
                k = 2 * c2 + par
                s = par
                wait_gathers(s)

                @pl.when(k + 1 < NCH)
                def _():
                    wait_in(1 - s)
                    double_indices(1 - s)
                    issue_gathers(1 - s)

                compute(k, s)

                @pl.when(k + 2 < NCH)
                def _():
                    issue_in(k + 2, s)

            return carry

        lax.fori_loop(0, NCH // 2, outer, 0)

    return pl.kernel(
        body,
        out_type=jax.ShapeDtypeStruct((B, D), jnp.float32),
        mesh=mesh,
        compiler_params=pltpu.CompilerParams(
            use_tc_tiling_on_sc=False, needs_layout_passes=False
        ),
        scratch_types=[
            pltpu.VMEM((C, H), jnp.int32),
            pltpu.VMEM((C, H), jnp.int32),
            pltpu.VMEM((C, H), jnp.float32),
            pltpu.VMEM((C, H), jnp.float32),
            pltpu.VMEM((C * H, D), jnp.bfloat16),
            pltpu.VMEM((C * H, D), jnp.bfloat16),
            pltpu.VMEM((C, D), jnp.float32),
            pltpu.SemaphoreType.DMA,
            pltpu.SemaphoreType.DMA,
            pltpu.SemaphoreType.DMA,
            pltpu.SemaphoreType.DMA,
            pltpu.SemaphoreType.DMA,
            pltpu.SemaphoreType.DMA,
        ],
    )


@jax.jit
def kernel(features, weights, table):
    B, H = features.shape
    V, D = table.shape
    table2 = jnp.pad(table.astype(jnp.bfloat16), ((0, 0), (0, 128 - D))).reshape(2 * V, D)
    return _make_kernel(B, H, V, D)(features.astype(jnp.int32), weights, table2)


# speedup vs baseline: 1.9591x; 1.9591x over previous
import jax
import jax.numpy as jnp
from jax import lax
from jax.experimental import pallas as pl
from jax.experimental.pallas import tpu as pltpu
from jax.experimental.pallas import tpu_sc as plsc

LANES = 16


def _make_kernel(B, H, V, D):
    info = plsc.get_sparse_core_info()
    NC, NS = info.num_cores, info.num_subcores
    NW = NC * NS
    EPW = B // NW
    C = 4
    NCH = EPW // C
    FL = C * H
    G0 = 128
    G1 = H - G0
    NLG = H // LANES
    TAIL = H - NLG * LANES
    TOFF = H - LANES
    ND = D // LANES

    mesh = plsc.VectorSubcoreMesh(core_axis_name="c", subcore_axis_name="s")

    def body(feat_hbm, wts_hbm, table_hbm, out_hbm,
             idx0, idx1, w0, w1, rows0, rows1, out_v,
             sem_f0, sem_f1, sem_w0, sem_w1, sem_g0, sem_g1):
        cid = lax.axis_index("c")
        sid = lax.axis_index("s")
        wid = sid * NC + cid
        row0 = wid * EPW

        slots = (
            (idx0, w0, rows0, sem_f0, sem_w0, sem_g0),
            (idx1, w1, rows1, sem_f1, sem_w1, sem_g1),
        )

        def issue_in(k, s):
            idx_v, w_v, _, sem_f, sem_w, _ = slots[s]
            r = row0 + k * C
            for i in range(C):
                pltpu.async_copy(feat_hbm.at[r + i], idx_v.at[pl.ds(i * H, H)], sem_f)
            pltpu.async_copy(wts_hbm.at[pl.ds(r, C)], w_v, sem_w)

        def wait_in(s):
            idx_v, w_v, _, sem_f, sem_w, _ = slots[s]
            for i in range(C):
                pltpu.make_async_copy(
                    feat_hbm.at[0], idx_v.at[pl.ds(i * H, H)], sem_f
                ).wait()
            pltpu.make_async_copy(wts_hbm.at[pl.ds(0, C)], w_v, sem_w).wait()

        def issue_gathers(s):
            idx_v, _, rows_v, _, _, sem_g = slots[s]
            for g in range(FL // G0):
                pltpu.async_copy(
                    table_hbm.at[idx_v.at[pl.ds(g * G0, G0)]],
                    rows_v.at[pl.ds(g * G0, G0)],
                    sem_g,
                )
            rem = FL % G0
            if rem:
                pltpu.async_copy(
                    table_hbm.at[idx_v.at[pl.ds(FL - rem, rem)]],
                    rows_v.at[pl.ds(FL - rem, rem)],
                    sem_g,
                )

        def double_indices(s):
            idx_v = slots[s][0]
            for g in range(FL // LANES):
                sl = pl.ds(g * LANES, LANES)
                idx_v[sl] = idx_v[sl] * 2

        def wait_gathers(s):
            _, _, rows_v, _, _, sem_g = slots[s]
            pltpu.make_async_copy(table_hbm.at[pl.ds(0, FL)], rows_v, sem_g).wait()

        def compute(k, s):
            _, w_v, rows_v, _, _, _ = slots[s]

            def elem_body(i, carry):
                base = i * H

                def lg_body(lg, accs):
                    off = lg * LANES
                    wv = w_v[i, pl.ds(off, LANES)]
                    accs = list(accs)
                    for j in range(LANES):
                        w_s = wv[j]
                        r = base + off + j
                        for dg in range(ND):
                            accs[dg] = accs[dg] + w_s * rows_v[r, pl.ds(dg * LANES, LANES)]
                    return tuple(accs)

                zero = jnp.zeros((LANES,), jnp.float32)
                accs = lax.fori_loop(0, NLG, lg_body, (zero,) * ND)
                accs = list(accs)
                wv = w_v[i, pl.ds(TOFF, LANES)]
                for j in range(TAIL):
                    w_s = wv[LANES - TAIL + j]
                    r = base + TOFF + LANES - TAIL + j
                    for dg in range(ND):
                        accs[dg] = accs[dg] + w_s * rows_v[r, pl.ds(dg * LANES, LANES)]
                for dg in range(ND):
                    out_v[i, pl.ds(dg * LANES, LANES)] = accs[dg]
                return carry

            lax.fori_loop(0, C, elem_body, 0)
            pltpu.sync_copy(out_v, out_hbm.at[pl.ds(row0 + k * C, C)])

        issue_in(0, 0)
        issue_in(1, 1)
        wait_in(0)
        double_indices(0)
        issue_gathers(0)

        def outer(c2, carry):
            for par in range(2):
                k = 2 * c2 + par
                s = par
                wait_gathers(s)

                @pl.when(k + 1 < NCH)
                def _():
                    wait_in(1 - s)
                    double_indices(1 - s)
                    issue_gathers(1 - s)

                compute(k, s)

                @pl.when(k + 2 < NCH)
                def _():
                    issue_in(k + 2, s)

            return carry

        lax.fori_loop(0, NCH // 2, outer, 0)

    return pl.kernel(
        body,
        out_type=jax.ShapeDtypeStruct((B, D), jnp.float32),
        mesh=mesh,
        compiler_params=pltpu.CompilerParams(use_tc_tiling_on_sc=False),
        scratch_types=[
            pltpu.VMEM((FL,), jnp.int32),
            pltpu.VMEM((FL,), jnp.int32),
            pltpu.VMEM((C, H), jnp.float32),
            pltpu.VMEM((C, H), jnp.float32),
            pltpu.VMEM((C * H, D), jnp.float32),
            pltpu.VMEM((C * H, D), jnp.float32),
            pltpu.VMEM((C, D), jnp.float32),
            pltpu.SemaphoreType.DMA,
            pltpu.SemaphoreType.DMA,
            pltpu.SemaphoreType.DMA,
            pltpu.SemaphoreType.DMA,
            pltpu.SemaphoreType.DMA,
            pltpu.SemaphoreType.DMA,
        ],
    )


@jax.jit
def kernel(features, weights, table):
    B, H = features.shape
    V, D = table.shape
    table2 = jnp.pad(table, ((0, 0), (0, 128 - D))).reshape(2 * V, D)
    return _make_kernel(B, H, V, D)(features.astype(jnp.int32), weights, table2)


# speedup vs baseline: 1.9602x; 1.0006x over previous
import jax
import jax.numpy as jnp
from jax import lax
from jax.experimental import pallas as pl
from jax.experimental.pallas import tpu as pltpu
from jax.experimental.pallas import tpu_sc as plsc

LANES = 16


def _make_kernel(B, H, V, D):
    info = plsc.get_sparse_core_info()
    NC, NS = info.num_cores, info.num_subcores
    NW = NC * NS
    EPW = B // NW
    C = 4
    NCH = EPW // C
    FL = C * H
    G0 = 128
    G1 = H - G0
    NLG = H // LANES
    TAIL = H - NLG * LANES
    TOFF = H - LANES
    ND = D // LANES

    mesh = plsc.VectorSubcoreMesh(core_axis_name="c", subcore_axis_name="s")

    def body(feat_hbm, wts_hbm, table_hbm, out_hbm,
             idx0, idx1, w0, w1, rows0, rows1, out_v,
             sem_f0, sem_f1, sem_w0, sem_w1, sem_g0, sem_g1):
        cid = lax.axis_index("c")
        sid = lax.axis_index("s")
        wid = sid * NC + cid
        row0 = wid * EPW

        slots = (
            (idx0, w0, rows0, sem_f0, sem_w0, sem_g0),
            (idx1, w1, rows1, sem_f1, sem_w1, sem_g1),
        )

        def issue_in(k, s):
            idx_v, w_v, _, sem_f, sem_w, _ = slots[s]
            r = row0 + k * C
            for i in range(C):
                pltpu.async_copy(feat_hbm.at[r + i], idx_v.at[pl.ds(i * H, H)], sem_f)
            pltpu.async_copy(wts_hbm.at[pl.ds(r, C)], w_v, sem_w)

        def wait_in(s):
            idx_v, w_v, _, sem_f, sem_w, _ = slots[s]
            for i in range(C):
                pltpu.make_async_copy(
                    feat_hbm.at[0], idx_v.at[pl.ds(i * H, H)], sem_f
                ).wait()
            pltpu.make_async_copy(wts_hbm.at[pl.ds(0, C)], w_v, sem_w).wait()

        def issue_gathers(s):
            idx_v, _, rows_v, _, _, sem_g = slots[s]
            for g in range(FL // G0):
                pltpu.async_copy(
                    table_hbm.at[idx_v.at[pl.ds(g * G0, G0)]],
                    rows_v.at[pl.ds(g * G0, G0)],
                    sem_g,
                )
            rem = FL % G0
            if rem:
                pltpu.async_copy(
                    table_hbm.at[idx_v.at[pl.ds(FL - rem, rem)]],
                    rows_v.at[pl.ds(FL - rem, rem)],
                    sem_g,
                )

        def double_indices(s):
            idx_v = slots[s][0]
            for g in range(FL // LANES):
                sl = pl.ds(g * LANES, LANES)
                idx_v[sl] = idx_v[sl] * 2

        def wait_gathers(s):
            _, _, rows_v, _, _, sem_g = slots[s]
            pltpu.make_async_copy(table_hbm.at[pl.ds(0, FL)], rows_v, sem_g).wait()

        def compute(k, s):
            _, w_v, rows_v, _, _, _ = slots[s]

            def elem_body(i, carry):
                base = i * H

                def lg_body(lg, accs):
                    off = lg * LANES
                    wv = w_v[i, pl.ds(off, LANES)]
                    accs = list(accs)
                    for j in range(LANES):
                        w_s = wv[j]
                        r = base + off + j
                        for dg in range(ND):
                            accs[dg] = accs[dg] + w_s * rows_v[r, pl.ds(dg * LANES, LANES)]
                    return tuple(accs)

                zero = jnp.zeros((LANES,), jnp.float32)
                accs = lax.fori_loop(0, NLG, lg_body, (zero,) * ND)
                accs = list(accs)
                wv = w_v[i, pl.ds(TOFF, LANES)]
                for j in range(TAIL):
                    w_s = wv[LANES - TAIL + j]
                    r = base + TOFF + LANES - TAIL + j
                    for dg in range(ND):
                        accs[dg] = accs[dg] + w_s * rows_v[r, pl.ds(dg * LANES, LANES)]
                for dg in range(ND):
                    out_v[i, pl.ds(dg * LANES, LANES)] = accs[dg]
                return carry

            lax.fori_loop(0, C, elem_body, 0)
            pltpu.sync_copy(out_v, out_hbm.at[pl.ds(row0 + k * C, C)])

        issue_in(0, 0)
        issue_in(1, 1)
        wait_in(0)
        double_indices(0)
        issue_gathers(0)

        def outer(c2, carry):
            for par in range(2):
                k = 2 * c2 + par
                s = par

                @pl.when(k + 1 < NCH)
                def _():
                    wait_in(1 - s)
                    double_indices(1 - s)
                    issue_gathers(1 - s)

                wait_gathers(s)
                compute(k, s)

                @pl.when(k + 2 < NCH)
                def _():
                    issue_in(k + 2, s)

            return carry

        lax.fori_loop(0, NCH // 2, outer, 0)

    return pl.kernel(
        body,
        out_type=jax.ShapeDtypeStruct((B, D), jnp.float32),
        mesh=mesh,
        compiler_params=pltpu.CompilerParams(use_tc_tiling_on_sc=False),
        scratch_types=[
            pltpu.VMEM((FL,), jnp.int32),
            pltpu.VMEM((FL,), jnp.int32),
            pltpu.VMEM((C, H), jnp.float32),
            pltpu.VMEM((C, H), jnp.float32),
            pltpu.VMEM((C * H, D), jnp.float32),
            pltpu.VMEM((C * H, D), jnp.float32),
            pltpu.VMEM((C, D), jnp.float32),
            pltpu.SemaphoreType.DMA,
            pltpu.SemaphoreType.DMA,
            pltpu.SemaphoreType.DMA,
            pltpu.SemaphoreType.DMA,
            pltpu.SemaphoreType.DMA,
            pltpu.SemaphoreType.DMA,
        ],
    )


@jax.jit
def kernel(features, weights, table):
    B, H = features.shape
    V, D = table.shape
    table2 = jnp.pad(table, ((0, 0), (0, 128 - D))).reshape(2 * V, D)
    return _make_kernel(B, H, V, D)(features.astype(jnp.int32), weights, table2)
